# baseline (speedup 1.0000x reference)
import functools

import jax
import jax.numpy as jnp
import numpy as np
from jax import lax
from jax.experimental import pallas as pl
from jax.experimental.pallas import tpu as pltpu
from jax.experimental.pallas import tpu_sc as plsc

N_NODES = 10000
N_EDGES = 320000
D_FEAT = 128
NPAD = 10240
NC, NS = 2, 16
NW = NC * NS
E_TILE = N_EDGES // NW
ROWS = 8
LANES = 128
GRP = LANES // 16
CHUNK = ROWS * LANES
N_CHUNK = 10
N_PAIR = N_CHUNK // 2
LAST_BASE = N_EDGES - N_CHUNK * CHUNK
N_SLICE = NPAD // NS

CUTOFF_DISTANCE = 4.0
CUTOFF_ONSET = 3.8
_D = 0.5 * (CUTOFF_DISTANCE - CUTOFF_ONSET)
_R = CUTOFF_DISTANCE - _D
_TSCALE = np.float32(np.pi / (2 * _D))

_TC_GRID = 5
_NODE_BLK = N_NODES // _TC_GRID


def _tc_pre_body(nf_ref, wsT_ref, bs_ref, wdT_ref, tbl_ref):
    x = nf_ref[...]
    es = jnp.exp(
        jnp.dot(x, wsT_ref[...], preferred_element_type=jnp.float32)
        + bs_ref[...])
    ed = jnp.exp(
        jnp.dot(x, wdT_ref[...], preferred_element_type=jnp.float32))
    tbl_ref[...] = jnp.concatenate([es, ed], axis=1)


_tc_pre = pl.pallas_call(
    _tc_pre_body,
    grid=(_TC_GRID,),
    in_specs=[
        pl.BlockSpec((_NODE_BLK, D_FEAT), lambda i: (i, 0)),
        pl.BlockSpec((D_FEAT, 4), lambda i: (0, 0)),
        pl.BlockSpec((1, 4), lambda i: (0, 0)),
        pl.BlockSpec((D_FEAT, 4), lambda i: (0, 0)),
    ],
    out_specs=pl.BlockSpec((_NODE_BLK, 8), lambda i: (i, 0)),
    out_shape=jax.ShapeDtypeStruct((N_NODES, 8), jnp.float32),
)


def _edge_scratch():
    return [
        pltpu.VMEM((CHUNK,), jnp.int32),
        pltpu.VMEM((CHUNK,), jnp.int32),
        pltpu.VMEM((CHUNK,), jnp.float32),
        pltpu.VMEM((CHUNK,), jnp.float32),
        pltpu.VMEM((ROWS, LANES), jnp.int32),
        pltpu.VMEM((ROWS, LANES), jnp.float32),
    ]


@functools.partial(
    pl.kernel,
    mesh=plsc.VectorSubcoreMesh(core_axis_name="c", subcore_axis_name="s"),
    compiler_params=pltpu.CompilerParams(needs_layout_passes=False),
    out_type=jax.ShapeDtypeStruct((NC, NPAD), jnp.float32),
    scratch_types=[
        pltpu.VMEM((N_NODES * 8,), jnp.float32),
        *_edge_scratch(),
        *_edge_scratch(),
        pltpu.VMEM((N_SLICE,), jnp.float32),
        pltpu.VMEM_SHARED((NPAD,), jnp.float32),
        pltpu.SemaphoreType.DMA,
        pltpu.SemaphoreType.DMA,
        pltpu.SemaphoreType.DMA,
        pltpu.SemaphoreType.DMA,
    ],
)
def _sc_edges(tbl_hbm, ei_hbm, bl_hbm, bo_hbm,
              out_hbm, tbl_v,
              srcA, dstA, blA, boA, dst2A, vpA,
              srcB, dstB, blB, boB, dst2B, vpB,
              zero_v, acc_sp, semT, semA, semB, semS):
    cid = lax.axis_index("c")
    sid = lax.axis_index("s")
    wid = sid * NC + cid
    base = jnp.where(wid == NW - 1, LAST_BASE, wid * E_TILE)
    setA = (srcA, dstA, blA, boA)
    setB = (srcB, dstB, blB, boB)

    def _hbm_slices(ch):
        off = base + ch * CHUNK
        return (ei_hbm.at[pl.ds(off, CHUNK)],
                ei_hbm.at[pl.ds(N_EDGES + off, CHUNK)],
                bl_hbm.at[pl.ds(off, CHUNK)],
                bo_hbm.at[pl.ds(off, CHUNK)])

    def _start_in(ch, bufs, sem):
        for hbm, buf in zip(_hbm_slices(ch), bufs):
            pltpu.async_copy(hbm, buf, sem)

    def _wait_in(bufs, sem):
        for hbm, buf in zip(_hbm_slices(0), bufs):
            pltpu.make_async_copy(hbm, buf, sem).wait()

    def _compute(src_v, dst_v, bl_v, bo_v, dst2_v, vp_v):
        for r in range(ROWS):
            for g in range(GRP):
                sl = pl.ds(r * LANES + g * 16, 16)
                sl2 = pl.ds(g * 16, 16)
                d = dst_v[sl]
                dst2_v[r, sl2] = d
                s8 = src_v[sl] * 8
                d8 = d * 8
                g0 = plsc.load_gather(tbl_v, [s8])
                g1 = plsc.load_gather(tbl_v, [s8 + 1])
                g2 = plsc.load_gather(tbl_v, [s8 + 2])
                g3 = plsc.load_gather(tbl_v, [s8 + 3])
                h0 = plsc.load_gather(tbl_v, [d8 + 4])
                h1 = plsc.load_gather(tbl_v, [d8 + 5])
                h2 = plsc.load_gather(tbl_v, [d8 + 6])
                h3 = plsc.load_gather(tbl_v, [d8 + 7])
                rr = bl_v[sl]
                t = (rr - _R) * _TSCALE
                t2 = t * t
                sn = t * (1.0 + t2 * (-1.0 / 6.0 + t2 * (
                    1.0 / 120.0 - t2 * (1.0 / 5040.0))))
                c = 0.5 - 0.5 * sn
                c = jnp.where(rr < _R - _D, 1.0, c)
                c = jnp.where(rr > _R + _D, 0.0, c)
                rep = c * g0 * h0 * jnp.exp(-g1 * h1 * rr)
                att = (c * bo_v[sl]) * g2 * h2 * jnp.exp(-g3 * h3 * rr)
                vp_v[r, sl2] = rep - att

    def _zero_groups(vp_v, row_groups):
        z = jnp.zeros((16,), jnp.float32)
        for r, g in row_groups:
            vp_v[r, pl.ds(g * 16, 16)] = z

    def _fire_scatter(vp_v, dst2_v):
        for r in range(ROWS):
            pltpu.async_copy(vp_v.at[r], acc_sp.at[dst2_v.at[r]],
                             semS, add=True)

    def _drain_scatter():
        for r in range(ROWS):
            pltpu.make_async_copy(vpB.at[r], acc_sp.at[dst2B.at[r]],
                                  semS).wait()

    tab = pltpu.async_copy(tbl_hbm, tbl_v, semT)
    _start_in(0, setA, semA)

    def _zero(i, carry):
        zero_v[pl.ds(i * 16, 16)] = jnp.zeros((16,), jnp.float32)
        return carry
    lax.fori_loop(0, N_SLICE // 16, _zero, 0)
    pltpu.sync_copy(zero_v, acc_sp.at[pl.ds(sid * N_SLICE, N_SLICE)])
    tab.wait()
    plsc.subcore_barrier()

    _zero_groups(vpB, [(0, g) for g in range(GRP)])
    z16i = jnp.zeros((16,), jnp.int32)
    for g in range(GRP):
        dst2B[0, pl.ds(g * 16, 16)] = z16i
    for r in range(ROWS):
        pltpu.async_copy(vpB.at[0], acc_sp.at[dst2B.at[0]], semS, add=True)

    _tail_groups = ([(ROWS - 2, g) for g in range(1, GRP)]
                    + [(ROWS - 1, g) for g in range(GRP)])

    def _pair(i, carry):
        _start_in(2 * i + 1, setB, semB)
        _wait_in(setA, semA)
        _compute(*setA, dst2A, vpA)

        @pl.when(jnp.logical_and(i == 0, wid == NW - 1))
        def _():
            _zero_groups(vpA, [(0, g) for g in range(GRP)]
                         + [(1, g) for g in range(GRP - 1)])
        _drain_scatter()
        _fire_scatter(vpA, dst2A)
        _wait_in(setB, semB)

        @pl.when(i < N_PAIR - 1)
        def _():
            _start_in(2 * i + 2, setA, semA)
        _compute(*setB, dst2B, vpB)

        @pl.when(jnp.logical_and(i == N_PAIR - 1, wid < NW - 1))
        def _():
            _zero_groups(vpB, _tail_groups)
        _drain_scatter()
        _fire_scatter(vpB, dst2B)
        return carry
    lax.fori_loop(0, N_PAIR, _pair, 0)

    _drain_scatter()
    plsc.subcore_barrier()
    pltpu.sync_copy(acc_sp.at[pl.ds(sid * N_SLICE, N_SLICE)],
                    out_hbm.at[cid, pl.ds(sid * N_SLICE, N_SLICE)])


def _tc_sum_body(x_ref, o_ref):
    o_ref[...] = x_ref[0, :] + x_ref[1, :]


_tc_sum = pl.pallas_call(
    _tc_sum_body,
    out_shape=jax.ShapeDtypeStruct((NPAD,), jnp.float32),
)


def kernel(node_features, bond_order, bondlength, edge_index, W_src, b_src, W_dst):
    tbl = _tc_pre(node_features, W_src.T, b_src.reshape(1, 4), W_dst.T)
    out2 = _sc_edges(tbl.reshape(-1), edge_index.reshape(-1),
                     bondlength, bond_order)
    return _tc_sum(out2)[:N_NODES]


# speedup vs baseline: 1.1938x; 1.1938x over previous
import functools

import jax
import jax.numpy as jnp
import numpy as np
from jax import lax
from jax.experimental import pallas as pl
from jax.experimental.pallas import tpu as pltpu
from jax.experimental.pallas import tpu_sc as plsc

N_NODES = 10000
N_EDGES = 320000
D_FEAT = 128
NPAD = 10240
NC, NS = 2, 16
NW = NC * NS
E_TILE = N_EDGES // NW
ROWS = 8
LANES = 128
GRP = LANES // 16
CHUNK = ROWS * LANES
N_CHUNK = 10
N_PAIR = N_CHUNK // 2
LAST_BASE = N_EDGES - N_CHUNK * CHUNK
N_SLICE = NPAD // NS

CUTOFF_DISTANCE = 4.0
CUTOFF_ONSET = 3.8
_D = 0.5 * (CUTOFF_DISTANCE - CUTOFF_ONSET)
_R = CUTOFF_DISTANCE - _D
_TSCALE = np.float32(np.pi / (2 * _D))

_TC_GRID = 5
_NODE_BLK = N_NODES // _TC_GRID


def _tc_pre_body(nf_ref, wsT_ref, bs_ref, wdT_ref, tbl_ref):
    x = nf_ref[...]
    es = jnp.exp(
        jnp.dot(x, wsT_ref[...], preferred_element_type=jnp.float32)
        + bs_ref[...])
    ed = jnp.exp(
        jnp.dot(x, wdT_ref[...], preferred_element_type=jnp.float32))
    tbl_ref[...] = jnp.concatenate([es, ed], axis=1)


_tc_pre = pl.pallas_call(
    _tc_pre_body,
    grid=(_TC_GRID,),
    in_specs=[
        pl.BlockSpec((_NODE_BLK, D_FEAT), lambda i: (i, 0)),
        pl.BlockSpec((D_FEAT, 4), lambda i: (0, 0)),
        pl.BlockSpec((1, 4), lambda i: (0, 0)),
        pl.BlockSpec((D_FEAT, 4), lambda i: (0, 0)),
    ],
    out_specs=pl.BlockSpec((_NODE_BLK, 8), lambda i: (i, 0)),
    out_shape=jax.ShapeDtypeStruct((N_NODES, 8), jnp.float32),
)


def _edge_scratch():
    return [
        pltpu.VMEM((CHUNK,), jnp.int32),
        pltpu.VMEM((CHUNK,), jnp.int32),
        pltpu.VMEM((CHUNK,), jnp.float32),
        pltpu.VMEM((CHUNK,), jnp.float32),
        pltpu.VMEM((ROWS, LANES), jnp.int32),
        pltpu.VMEM((ROWS, LANES), jnp.float32),
    ]


@functools.partial(
    pl.kernel,
    mesh=plsc.VectorSubcoreMesh(core_axis_name="c", subcore_axis_name="s"),
    compiler_params=pltpu.CompilerParams(needs_layout_passes=False),
    out_type=jax.ShapeDtypeStruct((NC, NPAD), jnp.float32),
    scratch_types=[
        pltpu.VMEM((N_NODES * 8,), jnp.float32),
        *_edge_scratch(),
        *_edge_scratch(),
        pltpu.VMEM((N_SLICE,), jnp.float32),
        pltpu.VMEM_SHARED((NPAD,), jnp.float32),
        pltpu.SemaphoreType.DMA,
        pltpu.SemaphoreType.DMA,
        pltpu.SemaphoreType.DMA,
        pltpu.SemaphoreType.DMA,
    ],
)
def _sc_edges(tbl_hbm, ei_hbm, bl_hbm, bo_hbm,
              out_hbm, tbl_v,
              srcA, dstA, blA, boA, dst2A, vpA,
              srcB, dstB, blB, boB, dst2B, vpB,
              zero_v, acc_sp, semT, semA, semB, semS):
    cid = lax.axis_index("c")
    sid = lax.axis_index("s")
    wid = sid * NC + cid
    base = jnp.where(wid == NW - 1, LAST_BASE, wid * E_TILE)
    setA = (srcA, dstA, blA, boA)
    setB = (srcB, dstB, blB, boB)

    def _hbm_slices(ch):
        off = base + ch * CHUNK
        return (ei_hbm.at[pl.ds(off, CHUNK)],
                ei_hbm.at[pl.ds(N_EDGES + off, CHUNK)],
                bl_hbm.at[pl.ds(off, CHUNK)],
                bo_hbm.at[pl.ds(off, CHUNK)])

    def _start_in(ch, bufs, sem):
        for hbm, buf in zip(_hbm_slices(ch), bufs):
            pltpu.async_copy(hbm, buf, sem)

    def _wait_in(bufs, sem):
        for hbm, buf in zip(_hbm_slices(0), bufs):
            pltpu.make_async_copy(hbm, buf, sem).wait()

    def _compute(src_v, dst_v, bl_v, bo_v, dst2_v, vp_v):
        for r in range(ROWS):
            for g in range(GRP):
                sl = pl.ds(r * LANES + g * 16, 16)
                sl2 = pl.ds(g * 16, 16)
                d = dst_v[sl]
                dst2_v[r, sl2] = d
                s8 = src_v[sl] * 8
                d8 = d * 8
                g0 = plsc.load_gather(tbl_v, [s8])
                g1 = plsc.load_gather(tbl_v, [s8 + 1])
                g2 = plsc.load_gather(tbl_v, [s8 + 2])
                g3 = plsc.load_gather(tbl_v, [s8 + 3])
                h0 = plsc.load_gather(tbl_v, [d8 + 4])
                h1 = plsc.load_gather(tbl_v, [d8 + 5])
                h2 = plsc.load_gather(tbl_v, [d8 + 6])
                h3 = plsc.load_gather(tbl_v, [d8 + 7])
                rr = bl_v[sl]
                t = (rr - _R) * _TSCALE
                t2 = t * t
                sn = t * (1.0 + t2 * (-1.0 / 6.0 + t2 * (
                    1.0 / 120.0 - t2 * (1.0 / 5040.0))))
                c = 0.5 - 0.5 * sn
                c = jnp.where(rr < _R - _D, 1.0, c)
                c = jnp.where(rr > _R + _D, 0.0, c)
                rep = c * g0 * h0 * jnp.exp(-g1 * h1 * rr)
                att = (c * bo_v[sl]) * g2 * h2 * jnp.exp(-g3 * h3 * rr)
                vp_v[r, sl2] = rep - att

    def _zero_groups(vp_v, row_groups):
        z = jnp.zeros((16,), jnp.float32)
        for r, g in row_groups:
            vp_v[r, pl.ds(g * 16, 16)] = z

    def _fire_scatter(vp_v, dst2_v):
        for r in range(ROWS):
            pltpu.async_copy(vp_v.at[r], acc_sp.at[dst2_v.at[r]],
                             semS, add=True)

    def _drain_scatter():
        for r in range(ROWS):
            pltpu.make_async_copy(vpB.at[r], acc_sp.at[dst2B.at[r]],
                                  semS).wait()

    tab = pltpu.async_copy(tbl_hbm, tbl_v, semT)
    _start_in(0, setA, semA)

    def _zero(i, carry):
        zero_v[pl.ds(i * 16, 16)] = jnp.zeros((16,), jnp.float32)
        return carry
    lax.fori_loop(0, N_SLICE // 16, _zero, 0)
    pltpu.sync_copy(zero_v, acc_sp.at[pl.ds(sid * N_SLICE, N_SLICE)])
    tab.wait()
    plsc.subcore_barrier()

    _zero_groups(vpB, [(0, g) for g in range(GRP)])
    iota16 = lax.iota(jnp.int32, 16)
    for g in range(GRP):
        dst2B[0, pl.ds(g * 16, 16)] = iota16 + (sid * LANES + g * 16)
    for r in range(ROWS):
        pltpu.async_copy(vpB.at[0], acc_sp.at[dst2B.at[0]], semS, add=True)

    _tail_groups = ([(ROWS - 2, g) for g in range(1, GRP)]
                    + [(ROWS - 1, g) for g in range(GRP)])

    def _pair(i, carry):
        _start_in(2 * i + 1, setB, semB)
        _wait_in(setA, semA)
        _compute(*setA, dst2A, vpA)

        @pl.when(jnp.logical_and(i == 0, wid == NW - 1))
        def _():
            _zero_groups(vpA, [(0, g) for g in range(GRP)]
                         + [(1, g) for g in range(GRP - 1)])
        _drain_scatter()
        _fire_scatter(vpA, dst2A)
        _wait_in(setB, semB)

        @pl.when(i < N_PAIR - 1)
        def _():
            _start_in(2 * i + 2, setA, semA)
        _compute(*setB, dst2B, vpB)

        @pl.when(jnp.logical_and(i == N_PAIR - 1, wid < NW - 1))
        def _():
            _zero_groups(vpB, _tail_groups)
        _drain_scatter()
        _fire_scatter(vpB, dst2B)
        return carry
    lax.fori_loop(0, N_PAIR, _pair, 0)

    _drain_scatter()
    plsc.subcore_barrier()
    pltpu.sync_copy(acc_sp.at[pl.ds(sid * N_SLICE, N_SLICE)],
                    out_hbm.at[cid, pl.ds(sid * N_SLICE, N_SLICE)])


def _tc_sum_body(x_ref, o_ref):
    o_ref[...] = x_ref[0, :] + x_ref[1, :]


_tc_sum = pl.pallas_call(
    _tc_sum_body,
    out_shape=jax.ShapeDtypeStruct((NPAD,), jnp.float32),
)


def kernel(node_features, bond_order, bondlength, edge_index, W_src, b_src, W_dst):
    tbl = _tc_pre(node_features, W_src.T, b_src.reshape(1, 4), W_dst.T)
    out2 = _sc_edges(tbl.reshape(-1), edge_index.reshape(-1),
                     bondlength, bond_order)
    return _tc_sum(out2)[:N_NODES]
